# Initial kernel scaffold; baseline (speedup 1.0000x reference)
import jax
import jax.numpy as jnp
from jax.experimental import pallas as pl


def kernel(x, edge_index, edge_attr, node_emb, edge_emb, conv_W, conv_b, head_W, head_b, ln_g, ln_b, ro_W, ro_b, rg_W, rg_b):
    raise NotImplementedError("write your pallas kernel here")


# speedup vs baseline: 1.5377x; 1.5377x over previous
import jax
import jax.numpy as jnp
from jax import lax
from jax.experimental import pallas as pl
from jax.experimental.pallas import tpu as pltpu
from jax.experimental.pallas import tpu_sc as plsc

N = 10000
E = 160000
H = 512
NUM_CONVS = 4
NODE_VOCAB = 118
EDGE_VOCAB = 5

LANES = 16
NS = 16
NC = 2
NPAD = 10240
SR = 160
PASS_ROWS = NC * NS * SR
NPASS = NPAD // PASS_ROWS
EB = 3200
NBLK = E // EB
NBLK_P = 56
NB_SCAN = EB // LANES
KB = 16
CAP = EB + 4 * KB
NW = NC * NS
TRASH_G = EDGE_VOCAB * N
BN = 1000
GRID_N = N // BN


def _embed_body(x_ref, emb_ref, o_ref):
    xb = x_ref[...]
    iota = lax.broadcasted_iota(jnp.int32, (BN, 128), 1).astype(jnp.float32)
    oh = jnp.where(iota == xb, 1.0, 0.0)
    o_ref[...] = jnp.dot(oh, emb_ref[...],
                         preferred_element_type=jnp.float32)


def _mbuild_body(h_ref, e_ref, m_ref):
    hb = h_ref[...]
    for k in range(EDGE_VOCAB):
        m_ref[k] = jnp.maximum(hb + e_ref[k:k + 1, :], 0.0)
    m_ref[EDGE_VOCAB] = jnp.zeros((BN, H), jnp.float32)


def _update_body(h_ref, a_ref, w_ref, b_ref, o_ref):
    t = h_ref[...] + a_ref[...]
    y = jnp.dot(t, w_ref[...], preferred_element_type=jnp.float32)
    y = y + b_ref[...]
    o_ref[...] = jnp.where(y >= 0, y, 0.01 * y)


def _head_body(skip_ref, h_ref, hw_ref, hb_ref, g_ref, bt_ref,
               rgw_ref, rgb_ref, row_ref, rob_ref,
               x_ref, z_ref, acc_ref):
    i = pl.program_id(0)
    hf = skip_ref[...] + h_ref[...]
    xm = jnp.dot(hf, hw_ref[...], preferred_element_type=jnp.float32)
    xm = xm + hb_ref[...]
    mu = jnp.mean(xm, axis=1, keepdims=True)
    xc = xm - mu
    var = jnp.mean(xc * xc, axis=1, keepdims=True)
    x_ref[...] = xc * lax.rsqrt(var + 1e-5) * g_ref[...] + bt_ref[...]

    bsum = jnp.sum(hf, axis=0, keepdims=True)

    @pl.when(i == 0)
    def _():
        acc_ref[...] = bsum

    @pl.when(i > 0)
    def _():
        acc_ref[...] = acc_ref[...] + bsum

    @pl.when(i == GRID_N - 1)
    def _():
        s = acc_ref[...]
        zc = jnp.concatenate([s, s * (1.0 / N)], axis=1)
        za = jnp.dot(zc, rgw_ref[...],
                     preferred_element_type=jnp.float32) + rgb_ref[...]
        zb = jnp.dot(zc, row_ref[...],
                     preferred_element_type=jnp.float32) + rob_ref[...]
        z_ref[...] = za * zb


def _sc_prep_body(src_ref, dst_ref, att_ref, ents_ref, cnts_ref,
                  se, sd, sa, b0, cbuf, esem):
    c = lax.axis_index("c")
    s = lax.axis_index("s")
    w = c * NS + s
    lane = lax.broadcasted_iota(jnp.int32, (LANES,), 0)

    def pbody(p, _p):
        sb = p * PASS_ROWS + w * SR

        def blkbody(blk, _b):
            boff = blk * EB
            d1 = pltpu.async_copy(src_ref.at[pl.ds(boff, EB)], se, esem)
            d2 = pltpu.async_copy(dst_ref.at[pl.ds(boff, EB)], sd, esem)
            d3 = pltpu.async_copy(att_ref.at[pl.ds(boff, EB)], sa, esem)
            d1.wait()
            d2.wait()
            d3.wait()

            def cbody(b, cc):
                off = b * LANES
                d = sd[pl.ds(off, LANES)]
                rel = d - sb
                m = (rel >= 0) & (rel < SR)
                g = sa[pl.ds(off, LANES)] * N + se[pl.ds(off, LANES)]
                v = g + rel * 65536
                inc = plsc.cumsum(m.astype(jnp.int32))
                plsc.store_scatter(b0, [cc + inc - 1], v, mask=m)
                tot = inc.at[jnp.full((LANES,), LANES - 1, jnp.int32)].get(
                    mode="promise_in_bounds")
                return cc + tot

            ccv = lax.fori_loop(0, NB_SCAN, cbody,
                                jnp.zeros((LANES,), jnp.int32))
            c0 = jnp.sum(jnp.where(lane == 0, ccv, 0))

            for q in range(4 * KB // LANES):
                tr = TRASH_G + w * (4 * KB) + q * LANES + lane + SR * 65536
                b0[pl.ds(c0 + q * LANES, LANES)] = tr

            nb2 = (c0 + 2 * KB - 1) // (2 * KB)
            plsc.store_scatter(cbuf, [jnp.full((LANES,), blk, jnp.int32)],
                               jnp.zeros((LANES,), jnp.int32) + nb2,
                               mask=lane == 0)
            pltpu.sync_copy(b0, ents_ref.at[p, w, blk])
            return _b

        lax.fori_loop(0, NBLK, blkbody, jnp.int32(0))
        pltpu.sync_copy(cbuf, cnts_ref.at[p, w])
        return _p

    lax.fori_loop(0, NPASS, pbody, jnp.int32(0))


def _sc_agg_body(mt_ref, ents_ref, cnts_ref, zz_ref, out_ref,
                 b0, rb, gvb, rowbuf, acc, cbuf,
                 esem, gsem0, gsem1):
    c = lax.axis_index("c")
    s = lax.axis_index("s")
    w = c * NS + s
    lane = lax.broadcasted_iota(jnp.int32, (LANES,), 0)

    def pbody(p, _p):
        sb = p * PASS_ROWS + w * SR
        pltpu.sync_copy(zz_ref, acc.at[pl.ds(0, SR)])
        pltpu.sync_copy(cnts_ref.at[p, w], cbuf)

        def blkbody(blk, _b):
            pltpu.sync_copy(ents_ref.at[p, w, blk], b0)
            cv = cbuf[pl.ds((blk // LANES) * LANES, LANES)]
            nb2 = jnp.sum(jnp.where(lane == blk % LANES, cv, 0))

            def build(j, par):
                for q in range(KB // LANES):
                    pv = b0[pl.ds(j * KB + q * LANES, LANES)]
                    gvb[par, pl.ds(q * LANES, LANES)] = pv & 65535
                    rb[par, pl.ds(q * LANES, LANES)] = (
                        lax.shift_right_logical(pv, 16))

            def fire(par, sem):
                pltpu.async_copy(mt_ref.at[gvb.at[par]],
                                 rowbuf.at[par], sem)

            def drain(par, sem):
                pltpu.make_async_copy(mt_ref.at[gvb.at[par]],
                                      rowbuf.at[par], sem).wait()

            def valu(par):
                for q in range(KB // LANES):
                    relv = rb[par, pl.ds(q * LANES, LANES)]
                    for e in range(LANES):
                        rel_e = jnp.sum(jnp.where(lane == e, relv, 0))
                        row = q * LANES + e
                        for k in range(H // LANES):
                            plsc.addupdate(
                                acc.at[rel_e, pl.ds(k * LANES, LANES)],
                                rowbuf[par, row, pl.ds(k * LANES, LANES)])

            @pl.when(nb2 > 0)
            def _():
                build(0, 0)
                fire(0, gsem0)

            def pair(i, _c):
                build(2 * i + 1, 1)
                fire(1, gsem1)
                drain(0, gsem0)
                valu(0)

                @pl.when(i + 1 < nb2)
                def _():
                    build(2 * i + 2, 0)
                    fire(0, gsem0)

                drain(1, gsem1)
                valu(1)
                return _c

            lax.fori_loop(0, nb2, pair, jnp.int32(0))
            return _b

        lax.fori_loop(0, NBLK, blkbody, jnp.int32(0))
        pltpu.sync_copy(acc.at[pl.ds(0, SR)], out_ref.at[pl.ds(sb, SR)])
        return _p

    lax.fori_loop(0, NPASS, pbody, jnp.int32(0))


def _embed(xf, emb_p):
    return pl.pallas_call(
        _embed_body,
        grid=(GRID_N,),
        in_specs=[pl.BlockSpec((BN, 1), lambda i: (i, 0)),
                  pl.BlockSpec((128, H), lambda i: (0, 0))],
        out_specs=pl.BlockSpec((BN, H), lambda i: (i, 0)),
        out_shape=jax.ShapeDtypeStruct((N, H), jnp.float32),
    )(xf, emb_p)


def _mbuild(h, e_p):
    return pl.pallas_call(
        _mbuild_body,
        grid=(GRID_N,),
        in_specs=[pl.BlockSpec((BN, H), lambda i: (i, 0)),
                  pl.BlockSpec((8, H), lambda i: (0, 0))],
        out_specs=pl.BlockSpec((EDGE_VOCAB + 1, BN, H), lambda i: (0, i, 0)),
        out_shape=jax.ShapeDtypeStruct((EDGE_VOCAB + 1, N, H), jnp.float32),
    )(h, e_p)


def _update(h, aggr, w, b):
    return pl.pallas_call(
        _update_body,
        grid=(GRID_N,),
        in_specs=[pl.BlockSpec((BN, H), lambda i: (i, 0)),
                  pl.BlockSpec((BN, H), lambda i: (i, 0)),
                  pl.BlockSpec((H, H), lambda i: (0, 0)),
                  pl.BlockSpec((1, H), lambda i: (0, 0))],
        out_specs=pl.BlockSpec((BN, H), lambda i: (i, 0)),
        out_shape=jax.ShapeDtypeStruct((N, H), jnp.float32),
    )(h, aggr, w, b)


def _head(skip, h, hw, hb, g, bt, rgw, rgb, row, rob):
    return pl.pallas_call(
        _head_body,
        grid=(GRID_N,),
        in_specs=[pl.BlockSpec((BN, H), lambda i: (i, 0)),
                  pl.BlockSpec((BN, H), lambda i: (i, 0)),
                  pl.BlockSpec((H, H), lambda i: (0, 0)),
                  pl.BlockSpec((1, H), lambda i: (0, 0)),
                  pl.BlockSpec((1, H), lambda i: (0, 0)),
                  pl.BlockSpec((1, H), lambda i: (0, 0)),
                  pl.BlockSpec((2 * H, H), lambda i: (0, 0)),
                  pl.BlockSpec((1, H), lambda i: (0, 0)),
                  pl.BlockSpec((2 * H, H), lambda i: (0, 0)),
                  pl.BlockSpec((1, H), lambda i: (0, 0))],
        out_specs=[pl.BlockSpec((BN, H), lambda i: (i, 0)),
                   pl.BlockSpec((1, H), lambda i: (0, 0))],
        out_shape=[jax.ShapeDtypeStruct((N, H), jnp.float32),
                   jax.ShapeDtypeStruct((1, H), jnp.float32)],
        scratch_shapes=[pltpu.VMEM((1, H), jnp.float32)],
    )(skip, h, hw, hb, g, bt, rgw, rgb, row, rob)


def _sc_mesh():
    return plsc.VectorSubcoreMesh(core_axis_name="c", subcore_axis_name="s",
                                  num_cores=NC, num_subcores=NS)


def _sc_prep(src, dst, att):
    f = pl.kernel(
        _sc_prep_body,
        out_type=[jax.ShapeDtypeStruct((NPASS, NW, NBLK, CAP), jnp.int32),
                  jax.ShapeDtypeStruct((NPASS, NW, NBLK_P), jnp.int32)],
        mesh=_sc_mesh(),
        compiler_params=pltpu.CompilerParams(needs_layout_passes=False),
        scratch_types=[
            pltpu.VMEM((EB,), jnp.int32),
            pltpu.VMEM((EB,), jnp.int32),
            pltpu.VMEM((EB,), jnp.int32),
            pltpu.VMEM((CAP,), jnp.int32),
            pltpu.VMEM((NBLK_P,), jnp.int32),
            pltpu.SemaphoreType.DMA,
        ],
    )
    return f(src, dst, att)


def _sc_agg(mt, ents, cnts, zz):
    f = pl.kernel(
        _sc_agg_body,
        out_type=jax.ShapeDtypeStruct((NPAD, H), jnp.float32),
        mesh=_sc_mesh(),
        compiler_params=pltpu.CompilerParams(needs_layout_passes=False),
        scratch_types=[
            pltpu.VMEM((CAP,), jnp.int32),
            pltpu.VMEM((2, KB), jnp.int32),
            pltpu.VMEM((2, KB), jnp.int32),
            pltpu.VMEM((2, KB, H), jnp.float32),
            pltpu.VMEM((SR + 4, H), jnp.float32),
            pltpu.VMEM((NBLK_P,), jnp.int32),
            pltpu.SemaphoreType.DMA,
            pltpu.SemaphoreType.DMA,
            pltpu.SemaphoreType.DMA,
        ],
    )
    return f(mt, ents, cnts, zz)


def kernel(x, edge_index, edge_attr, node_emb, edge_emb, conv_W, conv_b,
           head_W, head_b, ln_g, ln_b, ro_W, ro_b, rg_W, rg_b):
    xf = x.astype(jnp.float32).reshape(N, 1)
    emb_p = jnp.zeros((128, H), jnp.float32).at[:NODE_VOCAB].set(node_emb)
    e_p = jnp.zeros((8, H), jnp.float32).at[:EDGE_VOCAB].set(edge_emb)
    src = edge_index[0].astype(jnp.int32)
    dst = edge_index[1].astype(jnp.int32)
    att = edge_attr.astype(jnp.int32)
    zz = jnp.zeros((SR, H), jnp.float32)

    h = _embed(xf, emb_p)
    ents, cnts = _sc_prep(src, dst, att)
    skip = h
    for i in range(NUM_CONVS):
        mt = _mbuild(h, e_p).reshape((EDGE_VOCAB + 1) * N, H)
        aggr = _sc_agg(mt, ents, cnts, zz)
        h = _update(h, aggr, conv_W[i], conv_b[i].reshape(1, H))

    X, Z = _head(skip, h, head_W, head_b.reshape(1, H),
                 ln_g.reshape(1, H), ln_b.reshape(1, H),
                 rg_W, rg_b.reshape(1, H), ro_W, ro_b.reshape(1, H))
    return (X, Z)


# speedup vs baseline: 1.5396x; 1.0012x over previous
import jax
import jax.numpy as jnp
from jax import lax
from jax.experimental import pallas as pl
from jax.experimental.pallas import tpu as pltpu
from jax.experimental.pallas import tpu_sc as plsc

N = 10000
E = 160000
H = 512
NUM_CONVS = 4
NODE_VOCAB = 118
EDGE_VOCAB = 5

LANES = 16
NS = 16
NC = 2
NPAD = 10240
SR = 160
PASS_ROWS = NC * NS * SR
NPASS = NPAD // PASS_ROWS
EB = 3200
NBLK = E // EB
NBLK_P = 56
NB_SCAN = EB // LANES
KB = 16
CAP = EB + 4 * KB
NW = NC * NS
GB = 5
TRASH_G = EDGE_VOCAB * N
BN = 1000
GRID_N = N // BN


def _embed_body(x_ref, emb_ref, o_ref):
    xb = x_ref[...]
    iota = lax.broadcasted_iota(jnp.int32, (BN, 128), 1).astype(jnp.float32)
    oh = jnp.where(iota == xb, 1.0, 0.0)
    o_ref[...] = jnp.dot(oh, emb_ref[...],
                         preferred_element_type=jnp.float32)


def _mbuild_body(h_ref, e_ref, m_ref):
    hb = h_ref[...]
    for k in range(EDGE_VOCAB):
        m_ref[k] = jnp.maximum(hb + e_ref[k:k + 1, :], 0.0)
    m_ref[EDGE_VOCAB] = jnp.zeros((BN, H), jnp.float32)


def _update_body(h_ref, a_ref, w_ref, b_ref, o_ref):
    t = h_ref[...] + a_ref[...]
    y = jnp.dot(t, w_ref[...], preferred_element_type=jnp.float32)
    y = y + b_ref[...]
    o_ref[...] = jnp.where(y >= 0, y, 0.01 * y)


def _head_body(skip_ref, h_ref, hw_ref, hb_ref, g_ref, bt_ref,
               rgw_ref, rgb_ref, row_ref, rob_ref,
               x_ref, z_ref, acc_ref):
    i = pl.program_id(0)
    hf = skip_ref[...] + h_ref[...]
    xm = jnp.dot(hf, hw_ref[...], preferred_element_type=jnp.float32)
    xm = xm + hb_ref[...]
    mu = jnp.mean(xm, axis=1, keepdims=True)
    xc = xm - mu
    var = jnp.mean(xc * xc, axis=1, keepdims=True)
    x_ref[...] = xc * lax.rsqrt(var + 1e-5) * g_ref[...] + bt_ref[...]

    bsum = jnp.sum(hf, axis=0, keepdims=True)

    @pl.when(i == 0)
    def _():
        acc_ref[...] = bsum

    @pl.when(i > 0)
    def _():
        acc_ref[...] = acc_ref[...] + bsum

    @pl.when(i == GRID_N - 1)
    def _():
        s = acc_ref[...]
        zc = jnp.concatenate([s, s * (1.0 / N)], axis=1)
        za = jnp.dot(zc, rgw_ref[...],
                     preferred_element_type=jnp.float32) + rgb_ref[...]
        zb = jnp.dot(zc, row_ref[...],
                     preferred_element_type=jnp.float32) + rob_ref[...]
        z_ref[...] = za * zb


def _sc_prep_body(src_ref, dst_ref, att_ref, ents_ref, cnts_ref,
                  se, sd, sa, b0, cbuf, esem):
    c = lax.axis_index("c")
    s = lax.axis_index("s")
    w = c * NS + s
    lane = lax.broadcasted_iota(jnp.int32, (LANES,), 0)

    def pbody(p, _p):
        sb = p * PASS_ROWS + w * SR

        def blkbody(blk, _b):
            boff = blk * EB
            d1 = pltpu.async_copy(src_ref.at[pl.ds(boff, EB)], se, esem)
            d2 = pltpu.async_copy(dst_ref.at[pl.ds(boff, EB)], sd, esem)
            d3 = pltpu.async_copy(att_ref.at[pl.ds(boff, EB)], sa, esem)
            d1.wait()
            d2.wait()
            d3.wait()

            def cbody(b, cc):
                off = b * LANES
                d = sd[pl.ds(off, LANES)]
                rel = d - sb
                m = (rel >= 0) & (rel < SR)
                g = sa[pl.ds(off, LANES)] * N + se[pl.ds(off, LANES)]
                v = g + rel * 65536
                inc = plsc.cumsum(m.astype(jnp.int32))
                plsc.store_scatter(b0, [cc + inc - 1], v, mask=m)
                tot = inc.at[jnp.full((LANES,), LANES - 1, jnp.int32)].get(
                    mode="promise_in_bounds")
                return cc + tot

            ccv = lax.fori_loop(0, NB_SCAN, cbody,
                                jnp.zeros((LANES,), jnp.int32))
            c0 = jnp.sum(jnp.where(lane == 0, ccv, 0))

            for q in range(4 * KB // LANES):
                tr = TRASH_G + w * (4 * KB) + q * LANES + lane + SR * 65536
                b0[pl.ds(c0 + q * LANES, LANES)] = tr

            nb2 = (c0 + 2 * KB - 1) // (2 * KB)
            plsc.store_scatter(cbuf, [jnp.full((LANES,), blk, jnp.int32)],
                               jnp.zeros((LANES,), jnp.int32) + nb2,
                               mask=lane == 0)
            pltpu.sync_copy(b0, ents_ref.at[p, w, blk])
            return _b

        lax.fori_loop(0, NBLK, blkbody, jnp.int32(0))
        pltpu.sync_copy(cbuf, cnts_ref.at[p, w])
        return _p

    lax.fori_loop(0, NPASS, pbody, jnp.int32(0))


def _sc_agg_body(mt_ref, ents_ref, cnts_ref, zz_ref, out_ref,
                 b0, rb, gvb, rowbuf, acc, cbuf,
                 esem, gsem0, gsem1):
    c = lax.axis_index("c")
    s = lax.axis_index("s")
    w = c * NS + s
    lane = lax.broadcasted_iota(jnp.int32, (LANES,), 0)

    def build(sub, j, par):
        for q in range(KB // LANES):
            pv = b0[pl.ds(sub * CAP + j * KB + q * LANES, LANES)]
            gvb[par, pl.ds(q * LANES, LANES)] = pv & 65535
            rb[par, pl.ds(q * LANES, LANES)] = (
                lax.shift_right_logical(pv, 16))

    def fire(par, sem):
        pltpu.async_copy(mt_ref.at[gvb.at[par]], rowbuf.at[par], sem)

    def drain(par, sem):
        pltpu.make_async_copy(mt_ref.at[gvb.at[par]],
                              rowbuf.at[par], sem).wait()

    def valu(par):
        for q in range(KB // LANES):
            relv = rb[par, pl.ds(q * LANES, LANES)]
            for e in range(LANES):
                rel_e = jnp.sum(jnp.where(lane == e, relv, 0))
                row = q * LANES + e
                for k in range(H // LANES):
                    plsc.addupdate(
                        acc.at[rel_e, pl.ds(k * LANES, LANES)],
                        rowbuf[par, row, pl.ds(k * LANES, LANES)])

    def consume_block(sub, nb2):
        @pl.when(nb2 > 0)
        def _():
            build(sub, 0, 0)
            fire(0, gsem0)

        def pair(i, _c):
            build(sub, 2 * i + 1, 1)
            fire(1, gsem1)
            drain(0, gsem0)
            valu(0)

            @pl.when(i + 1 < nb2)
            def _():
                build(sub, 2 * i + 2, 0)
                fire(0, gsem0)

            drain(1, gsem1)
            valu(1)
            return _c

        lax.fori_loop(0, nb2, pair, jnp.int32(0))

    def pbody(p, _p):
        sb = p * PASS_ROWS + w * SR
        pltpu.sync_copy(zz_ref, acc.at[pl.ds(0, SR)])
        pltpu.sync_copy(cnts_ref.at[p, w], cbuf)

        def gbody(g, _b):
            pltpu.sync_copy(ents_ref.at[p, w, g], b0)

            def subbody(sub, _s):
                blk = g * GB + sub
                cv = cbuf[pl.ds((blk // LANES) * LANES, LANES)]
                nb2 = jnp.sum(jnp.where(lane == blk % LANES, cv, 0))
                consume_block(sub, nb2)
                return _s

            lax.fori_loop(0, GB, subbody, jnp.int32(0))
            return _b

        lax.fori_loop(0, NBLK // GB, gbody, jnp.int32(0))
        pltpu.sync_copy(acc.at[pl.ds(0, SR)], out_ref.at[pl.ds(sb, SR)])
        return _p

    lax.fori_loop(0, NPASS, pbody, jnp.int32(0))


def _embed(xf, emb_p):
    return pl.pallas_call(
        _embed_body,
        grid=(GRID_N,),
        in_specs=[pl.BlockSpec((BN, 1), lambda i: (i, 0)),
                  pl.BlockSpec((128, H), lambda i: (0, 0))],
        out_specs=pl.BlockSpec((BN, H), lambda i: (i, 0)),
        out_shape=jax.ShapeDtypeStruct((N, H), jnp.float32),
    )(xf, emb_p)


def _mbuild(h, e_p):
    return pl.pallas_call(
        _mbuild_body,
        grid=(GRID_N,),
        in_specs=[pl.BlockSpec((BN, H), lambda i: (i, 0)),
                  pl.BlockSpec((8, H), lambda i: (0, 0))],
        out_specs=pl.BlockSpec((EDGE_VOCAB + 1, BN, H), lambda i: (0, i, 0)),
        out_shape=jax.ShapeDtypeStruct((EDGE_VOCAB + 1, N, H), jnp.float32),
    )(h, e_p)


def _update(h, aggr, w, b):
    return pl.pallas_call(
        _update_body,
        grid=(GRID_N,),
        in_specs=[pl.BlockSpec((BN, H), lambda i: (i, 0)),
                  pl.BlockSpec((BN, H), lambda i: (i, 0)),
                  pl.BlockSpec((H, H), lambda i: (0, 0)),
                  pl.BlockSpec((1, H), lambda i: (0, 0))],
        out_specs=pl.BlockSpec((BN, H), lambda i: (i, 0)),
        out_shape=jax.ShapeDtypeStruct((N, H), jnp.float32),
    )(h, aggr, w, b)


def _head(skip, h, hw, hb, g, bt, rgw, rgb, row, rob):
    return pl.pallas_call(
        _head_body,
        grid=(GRID_N,),
        in_specs=[pl.BlockSpec((BN, H), lambda i: (i, 0)),
                  pl.BlockSpec((BN, H), lambda i: (i, 0)),
                  pl.BlockSpec((H, H), lambda i: (0, 0)),
                  pl.BlockSpec((1, H), lambda i: (0, 0)),
                  pl.BlockSpec((1, H), lambda i: (0, 0)),
                  pl.BlockSpec((1, H), lambda i: (0, 0)),
                  pl.BlockSpec((2 * H, H), lambda i: (0, 0)),
                  pl.BlockSpec((1, H), lambda i: (0, 0)),
                  pl.BlockSpec((2 * H, H), lambda i: (0, 0)),
                  pl.BlockSpec((1, H), lambda i: (0, 0))],
        out_specs=[pl.BlockSpec((BN, H), lambda i: (i, 0)),
                   pl.BlockSpec((1, H), lambda i: (0, 0))],
        out_shape=[jax.ShapeDtypeStruct((N, H), jnp.float32),
                   jax.ShapeDtypeStruct((1, H), jnp.float32)],
        scratch_shapes=[pltpu.VMEM((1, H), jnp.float32)],
    )(skip, h, hw, hb, g, bt, rgw, rgb, row, rob)


def _sc_mesh():
    return plsc.VectorSubcoreMesh(core_axis_name="c", subcore_axis_name="s",
                                  num_cores=NC, num_subcores=NS)


def _sc_prep(src, dst, att):
    f = pl.kernel(
        _sc_prep_body,
        out_type=[jax.ShapeDtypeStruct((NPASS, NW, NBLK, CAP), jnp.int32),
                  jax.ShapeDtypeStruct((NPASS, NW, NBLK_P), jnp.int32)],
        mesh=_sc_mesh(),
        compiler_params=pltpu.CompilerParams(needs_layout_passes=False),
        scratch_types=[
            pltpu.VMEM((EB,), jnp.int32),
            pltpu.VMEM((EB,), jnp.int32),
            pltpu.VMEM((EB,), jnp.int32),
            pltpu.VMEM((CAP,), jnp.int32),
            pltpu.VMEM((NBLK_P,), jnp.int32),
            pltpu.SemaphoreType.DMA,
        ],
    )
    return f(src, dst, att)


def _sc_agg(mt, ents, cnts, zz):
    ents = ents.reshape(NPASS, NW, NBLK // GB, GB * CAP)
    f = pl.kernel(
        _sc_agg_body,
        out_type=jax.ShapeDtypeStruct((NPAD, H), jnp.float32),
        mesh=_sc_mesh(),
        compiler_params=pltpu.CompilerParams(needs_layout_passes=False),
        scratch_types=[
            pltpu.VMEM((GB * CAP,), jnp.int32),
            pltpu.VMEM((2, KB), jnp.int32),
            pltpu.VMEM((2, KB), jnp.int32),
            pltpu.VMEM((2, KB, H), jnp.float32),
            pltpu.VMEM((SR + 4, H), jnp.float32),
            pltpu.VMEM((NBLK_P,), jnp.int32),
            pltpu.SemaphoreType.DMA,
            pltpu.SemaphoreType.DMA,
            pltpu.SemaphoreType.DMA,
        ],
    )
    return f(mt, ents, cnts, zz)


def kernel(x, edge_index, edge_attr, node_emb, edge_emb, conv_W, conv_b,
           head_W, head_b, ln_g, ln_b, ro_W, ro_b, rg_W, rg_b):
    xf = x.astype(jnp.float32).reshape(N, 1)
    emb_p = jnp.zeros((128, H), jnp.float32).at[:NODE_VOCAB].set(node_emb)
    e_p = jnp.zeros((8, H), jnp.float32).at[:EDGE_VOCAB].set(edge_emb)
    src = edge_index[0].astype(jnp.int32)
    dst = edge_index[1].astype(jnp.int32)
    att = edge_attr.astype(jnp.int32)
    zz = jnp.zeros((SR, H), jnp.float32)

    h = _embed(xf, emb_p)
    ents, cnts = _sc_prep(src, dst, att)
    skip = h
    for i in range(NUM_CONVS):
        mt = _mbuild(h, e_p).reshape((EDGE_VOCAB + 1) * N, H)
        aggr = _sc_agg(mt, ents, cnts, zz)
        h = _update(h, aggr, conv_W[i], conv_b[i].reshape(1, H))

    X, Z = _head(skip, h, head_W, head_b.reshape(1, H),
                 ln_g.reshape(1, H), ln_b.reshape(1, H),
                 rg_W, rg_b.reshape(1, H), ro_W, ro_b.reshape(1, H))
    return (X, Z)


# speedup vs baseline: 1.5998x; 1.0391x over previous
import jax
import jax.numpy as jnp
from jax import lax
from jax.experimental import pallas as pl
from jax.experimental.pallas import tpu as pltpu
from jax.experimental.pallas import tpu_sc as plsc

N = 10000
E = 160000
H = 512
NUM_CONVS = 4
NODE_VOCAB = 118
EDGE_VOCAB = 5

LANES = 16
NS = 16
NC = 2
NPAD = 10240
SR = 160
PASS_ROWS = NC * NS * SR
NPASS = NPAD // PASS_ROWS
EB = 3200
NBLK = E // EB
NBLK_P = 56
NB_SCAN = EB // LANES
KB = 16
CAP = EB + 4 * KB
NW = NC * NS
GB = 5
TRASH_G = EDGE_VOCAB * N
BN = 1000
GRID_N = N // BN


def _embed_body(x_ref, emb_ref, o_ref):
    xb = x_ref[...]
    iota = lax.broadcasted_iota(jnp.int32, (BN, 128), 1).astype(jnp.float32)
    oh = jnp.where(iota == xb, 1.0, 0.0)
    o_ref[...] = jnp.dot(oh, emb_ref[...],
                         preferred_element_type=jnp.float32)


def _mbuild_body(h_ref, e_ref, m_ref):
    hb = h_ref[...]
    for k in range(EDGE_VOCAB):
        m_ref[k] = jnp.maximum(hb + e_ref[k:k + 1, :], 0.0)
    m_ref[EDGE_VOCAB] = jnp.zeros((BN, H), jnp.float32)


def _update_body(h_ref, a_ref, w_ref, b_ref, o_ref):
    t = h_ref[...] + a_ref[...]
    y = jnp.dot(t, w_ref[...], preferred_element_type=jnp.float32)
    y = y + b_ref[...]
    o_ref[...] = jnp.where(y >= 0, y, 0.01 * y)


def _head_body(skip_ref, h_ref, hw_ref, hb_ref, g_ref, bt_ref,
               rgw_ref, rgb_ref, row_ref, rob_ref,
               x_ref, z_ref, acc_ref):
    i = pl.program_id(0)
    hf = skip_ref[...] + h_ref[...]
    xm = jnp.dot(hf, hw_ref[...], preferred_element_type=jnp.float32)
    xm = xm + hb_ref[...]
    mu = jnp.mean(xm, axis=1, keepdims=True)
    xc = xm - mu
    var = jnp.mean(xc * xc, axis=1, keepdims=True)
    x_ref[...] = xc * lax.rsqrt(var + 1e-5) * g_ref[...] + bt_ref[...]

    bsum = jnp.sum(hf, axis=0, keepdims=True)

    @pl.when(i == 0)
    def _():
        acc_ref[...] = bsum

    @pl.when(i > 0)
    def _():
        acc_ref[...] = acc_ref[...] + bsum

    @pl.when(i == GRID_N - 1)
    def _():
        s = acc_ref[...]
        zc = jnp.concatenate([s, s * (1.0 / N)], axis=1)
        za = jnp.dot(zc, rgw_ref[...],
                     preferred_element_type=jnp.float32) + rgb_ref[...]
        zb = jnp.dot(zc, row_ref[...],
                     preferred_element_type=jnp.float32) + rob_ref[...]
        z_ref[...] = za * zb


def _sc_prep_body(src_ref, dst_ref, att_ref, ents_ref, cnts_ref,
                  se, sd, sa, b0, cbuf, esem):
    c = lax.axis_index("c")
    s = lax.axis_index("s")
    w = c * NS + s
    lane = lax.broadcasted_iota(jnp.int32, (LANES,), 0)

    def pbody(p, _p):
        sb = p * PASS_ROWS + w * SR

        def blkbody(blk, _b):
            boff = blk * EB
            d1 = pltpu.async_copy(src_ref.at[pl.ds(boff, EB)], se, esem)
            d2 = pltpu.async_copy(dst_ref.at[pl.ds(boff, EB)], sd, esem)
            d3 = pltpu.async_copy(att_ref.at[pl.ds(boff, EB)], sa, esem)
            d1.wait()
            d2.wait()
            d3.wait()

            def cbody(b, cc):
                off = b * LANES
                d = sd[pl.ds(off, LANES)]
                rel = d - sb
                m = (rel >= 0) & (rel < SR)
                g = sa[pl.ds(off, LANES)] * N + se[pl.ds(off, LANES)]
                v = g + rel * 65536
                inc = plsc.cumsum(m.astype(jnp.int32))
                plsc.store_scatter(b0, [cc + inc - 1], v, mask=m)
                tot = inc.at[jnp.full((LANES,), LANES - 1, jnp.int32)].get(
                    mode="promise_in_bounds")
                return cc + tot

            ccv = lax.fori_loop(0, NB_SCAN, cbody,
                                jnp.zeros((LANES,), jnp.int32))
            c0 = jnp.sum(jnp.where(lane == 0, ccv, 0))

            for q in range(4 * KB // LANES):
                tr = TRASH_G + w * (4 * KB) + q * LANES + lane + SR * 65536
                b0[pl.ds(c0 + q * LANES, LANES)] = tr

            nb2 = (c0 + 2 * KB - 1) // (2 * KB)
            plsc.store_scatter(cbuf, [jnp.full((LANES,), blk, jnp.int32)],
                               jnp.zeros((LANES,), jnp.int32) + nb2,
                               mask=lane == 0)
            pltpu.sync_copy(b0, ents_ref.at[p, w, blk])
            return _b

        lax.fori_loop(0, NBLK, blkbody, jnp.int32(0))
        pltpu.sync_copy(cbuf, cnts_ref.at[p, w])
        return _p

    lax.fori_loop(0, NPASS, pbody, jnp.int32(0))


def _sc_agg_body(mt_ref, ents_ref, cnts_ref, zz_ref, out_ref,
                 b0, rb, gvb, rowbuf, acc, cbuf,
                 esem, gsem0, gsem1):
    c = lax.axis_index("c")
    s = lax.axis_index("s")
    w = c * NS + s
    lane = lax.broadcasted_iota(jnp.int32, (LANES,), 0)

    def build(sub, j, par):
        for q in range(KB // LANES):
            pv = b0[pl.ds(sub * CAP + j * KB + q * LANES, LANES)]
            gvb[par, pl.ds(q * LANES, LANES)] = pv & 65535
            rb[par, pl.ds(q * LANES, LANES)] = (
                lax.shift_right_logical(pv, 16))

    def fire(par, sem):
        pltpu.async_copy(mt_ref.at[gvb.at[par]], rowbuf.at[par], sem)

    def drain(par, sem):
        pltpu.make_async_copy(mt_ref.at[gvb.at[par]],
                              rowbuf.at[par], sem).wait()

    def valu(par):
        for q in range(KB // LANES):
            relv = rb[par, pl.ds(q * LANES, LANES)]
            for e in range(LANES):
                dsplat = relv.at[jnp.full((LANES,), e, jnp.int32)].get(
                    mode="promise_in_bounds")
                base = dsplat * H + lane
                row = q * LANES + e
                for k in range(H // LANES):
                    vals = rowbuf[par, row, pl.ds(k * LANES, LANES)]
                    plsc.addupdate_scatter(acc, [base + k * LANES], vals)

    def consume_block(sub, nb2):
        @pl.when(nb2 > 0)
        def _():
            build(sub, 0, 0)
            fire(0, gsem0)

        def pair(i, _c):
            build(sub, 2 * i + 1, 1)
            fire(1, gsem1)
            drain(0, gsem0)
            valu(0)

            @pl.when(i + 1 < nb2)
            def _():
                build(sub, 2 * i + 2, 0)
                fire(0, gsem0)

            drain(1, gsem1)
            valu(1)
            return _c

        lax.fori_loop(0, nb2, pair, jnp.int32(0))

    def pbody(p, _p):
        sb = p * PASS_ROWS + w * SR
        pltpu.sync_copy(zz_ref, acc.at[pl.ds(0, SR * H)])
        pltpu.sync_copy(cnts_ref.at[p, w], cbuf)

        def gbody(g, _b):
            pltpu.sync_copy(ents_ref.at[p, w, g], b0)

            def subbody(sub, _s):
                blk = g * GB + sub
                cv = cbuf[pl.ds((blk // LANES) * LANES, LANES)]
                nb2 = jnp.sum(jnp.where(lane == blk % LANES, cv, 0))
                consume_block(sub, nb2)
                return _s

            lax.fori_loop(0, GB, subbody, jnp.int32(0))
            return _b

        lax.fori_loop(0, NBLK // GB, gbody, jnp.int32(0))
        pltpu.sync_copy(acc.at[pl.ds(0, SR * H)],
                        out_ref.at[pl.ds(sb * H, SR * H)])
        return _p

    lax.fori_loop(0, NPASS, pbody, jnp.int32(0))


def _embed(xf, emb_p):
    return pl.pallas_call(
        _embed_body,
        grid=(GRID_N,),
        in_specs=[pl.BlockSpec((BN, 1), lambda i: (i, 0)),
                  pl.BlockSpec((128, H), lambda i: (0, 0))],
        out_specs=pl.BlockSpec((BN, H), lambda i: (i, 0)),
        out_shape=jax.ShapeDtypeStruct((N, H), jnp.float32),
    )(xf, emb_p)


def _mbuild(h, e_p):
    return pl.pallas_call(
        _mbuild_body,
        grid=(GRID_N,),
        in_specs=[pl.BlockSpec((BN, H), lambda i: (i, 0)),
                  pl.BlockSpec((8, H), lambda i: (0, 0))],
        out_specs=pl.BlockSpec((EDGE_VOCAB + 1, BN, H), lambda i: (0, i, 0)),
        out_shape=jax.ShapeDtypeStruct((EDGE_VOCAB + 1, N, H), jnp.float32),
    )(h, e_p)


def _update(h, aggr, w, b):
    return pl.pallas_call(
        _update_body,
        grid=(GRID_N,),
        in_specs=[pl.BlockSpec((BN, H), lambda i: (i, 0)),
                  pl.BlockSpec((BN, H), lambda i: (i, 0)),
                  pl.BlockSpec((H, H), lambda i: (0, 0)),
                  pl.BlockSpec((1, H), lambda i: (0, 0))],
        out_specs=pl.BlockSpec((BN, H), lambda i: (i, 0)),
        out_shape=jax.ShapeDtypeStruct((N, H), jnp.float32),
    )(h, aggr, w, b)


def _head(skip, h, hw, hb, g, bt, rgw, rgb, row, rob):
    return pl.pallas_call(
        _head_body,
        grid=(GRID_N,),
        in_specs=[pl.BlockSpec((BN, H), lambda i: (i, 0)),
                  pl.BlockSpec((BN, H), lambda i: (i, 0)),
                  pl.BlockSpec((H, H), lambda i: (0, 0)),
                  pl.BlockSpec((1, H), lambda i: (0, 0)),
                  pl.BlockSpec((1, H), lambda i: (0, 0)),
                  pl.BlockSpec((1, H), lambda i: (0, 0)),
                  pl.BlockSpec((2 * H, H), lambda i: (0, 0)),
                  pl.BlockSpec((1, H), lambda i: (0, 0)),
                  pl.BlockSpec((2 * H, H), lambda i: (0, 0)),
                  pl.BlockSpec((1, H), lambda i: (0, 0))],
        out_specs=[pl.BlockSpec((BN, H), lambda i: (i, 0)),
                   pl.BlockSpec((1, H), lambda i: (0, 0))],
        out_shape=[jax.ShapeDtypeStruct((N, H), jnp.float32),
                   jax.ShapeDtypeStruct((1, H), jnp.float32)],
        scratch_shapes=[pltpu.VMEM((1, H), jnp.float32)],
    )(skip, h, hw, hb, g, bt, rgw, rgb, row, rob)


def _sc_mesh():
    return plsc.VectorSubcoreMesh(core_axis_name="c", subcore_axis_name="s",
                                  num_cores=NC, num_subcores=NS)


def _sc_prep(src, dst, att):
    f = pl.kernel(
        _sc_prep_body,
        out_type=[jax.ShapeDtypeStruct((NPASS, NW, NBLK, CAP), jnp.int32),
                  jax.ShapeDtypeStruct((NPASS, NW, NBLK_P), jnp.int32)],
        mesh=_sc_mesh(),
        compiler_params=pltpu.CompilerParams(needs_layout_passes=False),
        scratch_types=[
            pltpu.VMEM((EB,), jnp.int32),
            pltpu.VMEM((EB,), jnp.int32),
            pltpu.VMEM((EB,), jnp.int32),
            pltpu.VMEM((CAP,), jnp.int32),
            pltpu.VMEM((NBLK_P,), jnp.int32),
            pltpu.SemaphoreType.DMA,
        ],
    )
    return f(src, dst, att)


def _sc_agg(mt, ents, cnts, zz):
    ents = ents.reshape(NPASS, NW, NBLK // GB, GB * CAP)
    f = pl.kernel(
        _sc_agg_body,
        out_type=jax.ShapeDtypeStruct((NPAD * H,), jnp.float32),
        mesh=_sc_mesh(),
        compiler_params=pltpu.CompilerParams(needs_layout_passes=False),
        scratch_types=[
            pltpu.VMEM((GB * CAP,), jnp.int32),
            pltpu.VMEM((2, KB), jnp.int32),
            pltpu.VMEM((2, KB), jnp.int32),
            pltpu.VMEM((2, KB, H), jnp.float32),
            pltpu.VMEM(((SR + 4) * H,), jnp.float32),
            pltpu.VMEM((NBLK_P,), jnp.int32),
            pltpu.SemaphoreType.DMA,
            pltpu.SemaphoreType.DMA,
            pltpu.SemaphoreType.DMA,
        ],
    )
    return f(mt, ents, cnts, zz).reshape(NPAD, H)


def kernel(x, edge_index, edge_attr, node_emb, edge_emb, conv_W, conv_b,
           head_W, head_b, ln_g, ln_b, ro_W, ro_b, rg_W, rg_b):
    xf = x.astype(jnp.float32).reshape(N, 1)
    emb_p = jnp.zeros((128, H), jnp.float32).at[:NODE_VOCAB].set(node_emb)
    e_p = jnp.zeros((8, H), jnp.float32).at[:EDGE_VOCAB].set(edge_emb)
    src = edge_index[0].astype(jnp.int32)
    dst = edge_index[1].astype(jnp.int32)
    att = edge_attr.astype(jnp.int32)
    zz = jnp.zeros((SR * H,), jnp.float32)

    h = _embed(xf, emb_p)
    ents, cnts = _sc_prep(src, dst, att)
    skip = h
    for i in range(NUM_CONVS):
        mt = _mbuild(h, e_p).reshape((EDGE_VOCAB + 1) * N, H)
        aggr = _sc_agg(mt, ents, cnts, zz)
        h = _update(h, aggr, conv_W[i], conv_b[i].reshape(1, H))

    X, Z = _head(skip, h, head_W, head_b.reshape(1, H),
                 ln_g.reshape(1, H), ln_b.reshape(1, H),
                 rg_W, rg_b.reshape(1, H), ro_W, ro_b.reshape(1, H))
    return (X, Z)
